# Initial kernel scaffold; baseline (speedup 1.0000x reference)
import jax
import jax.numpy as jnp
from jax.experimental import pallas as pl


def kernel(x, edge_index, W1_l, b1_l, W1_r, W2_l, b2_l, W2_r):
    raise NotImplementedError("write your pallas kernel here")


# speedup vs baseline: 2.6907x; 2.6907x over previous
import jax
import jax.numpy as jnp
from jax import lax
from jax.experimental import pallas as pl
from jax.experimental.pallas import tpu as pltpu
from jax.experimental.pallas import tpu_sc as plsc

N = 10000
E = 320000
D = 128

NC = 2
NS = 16
NW = NC * NS
CHUNK = 128
K = -(-E // (NW * CHUNK))
EPT = K * CHUNK
E_PAD = NW * EPT
ROWS_PER_TILE = 640
NPAD = NS * ROWS_PER_TILE
F32 = jnp.float32
I32 = jnp.int32


def _segsum_call(with_deg):
    mesh = plsc.VectorSubcoreMesh(core_axis_name="c", subcore_axis_name="s")
    out_type = [jax.ShapeDtypeStruct((NC * NPAD, D), F32)]
    scratch = [
        pltpu.VMEM((1, CHUNK), I32),
        pltpu.VMEM((1, CHUNK), I32),
        pltpu.VMEM((CHUNK, D), F32),
        pltpu.VMEM_SHARED((NPAD, D), F32),
        pltpu.SemaphoreType.DMA,
    ]
    if with_deg:
        out_type.append(jax.ShapeDtypeStruct((NC * NPAD, D), F32))

    def body(rows_hbm, src_hbm, dst_hbm, zrows_hbm, orows_hbm, agg_out, *rest):
        if with_deg:
            deg_out, src_v, dst_v, rows_v, agg_sh, sem = rest
        else:
            src_v, dst_v, rows_v, agg_sh, sem = rest
        c = lax.axis_index("c")
        s = lax.axis_index("s")
        wid = s * NC + c

        def fill(t, _):
            pltpu.sync_copy(rows_v,
                            agg_sh.at[pl.ds(s * ROWS_PER_TILE + t * CHUNK, CHUNK)])
            return 0

        def wb(out):
            def step(t, _):
                r0 = s * ROWS_PER_TILE + t * CHUNK
                pltpu.sync_copy(agg_sh.at[pl.ds(r0, CHUNK)],
                                out.at[pl.ds(c * NPAD + r0, CHUNK)])
                return 0
            lax.fori_loop(0, ROWS_PER_TILE // CHUNK, step, 0)

        if with_deg:
            pltpu.sync_copy(zrows_hbm, rows_v)
            lax.fori_loop(0, ROWS_PER_TILE // CHUNK, fill, 0)
            pltpu.sync_copy(orows_hbm, rows_v)
            plsc.subcore_barrier()

            def deg_chunk(j, _):
                row = wid * K + j
                pltpu.sync_copy(dst_hbm.at[pl.ds(row, 1)], dst_v)
                pltpu.sync_copy(rows_v, agg_sh.at[dst_v.at[0]], add=True)
                return 0
            lax.fori_loop(0, K, deg_chunk, 0)
            plsc.subcore_barrier()
            wb(deg_out)
            plsc.subcore_barrier()

        pltpu.sync_copy(zrows_hbm, rows_v)
        lax.fori_loop(0, ROWS_PER_TILE // CHUNK, fill, 0)
        plsc.subcore_barrier()

        def chunk_body(j, _):
            row = wid * K + j
            pltpu.sync_copy(src_hbm.at[pl.ds(row, 1)], src_v)
            pltpu.sync_copy(dst_hbm.at[pl.ds(row, 1)], dst_v)
            pltpu.async_copy(rows_hbm.at[src_v.at[0]], rows_v, sem).wait()
            pltpu.sync_copy(rows_v, agg_sh.at[dst_v.at[0]], add=True)
            return 0
        lax.fori_loop(0, K, chunk_body, 0)
        plsc.subcore_barrier()
        wb(agg_out)

    return pl.kernel(body, out_type=out_type, mesh=mesh, scratch_types=scratch)


def _dense_call(relu):
    BLK = 1024
    grid = NPAD // BLK
    half = NPAD // BLK

    def body(agg0_ref, agg1_ref, deg0_ref, deg1_ref, x_ref, wl_ref, wr_ref,
             b_ref, out_ref):
        agg = agg0_ref[...] + agg1_ref[...]
        deg = deg0_ref[:, :1] + deg1_ref[:, :1]
        mean = agg / jnp.maximum(deg, 1.0)
        h = (jnp.dot(mean, wl_ref[...], preferred_element_type=F32)
             + jnp.dot(x_ref[...], wr_ref[...], preferred_element_type=F32)
             + b_ref[...])
        out_ref[...] = jnp.maximum(h, 0.0) if relu else h

    return pl.pallas_call(
        body,
        grid=(grid,),
        in_specs=[
            pl.BlockSpec((BLK, D), lambda i: (i, 0)),
            pl.BlockSpec((BLK, D), lambda i: (half + i, 0)),
            pl.BlockSpec((BLK, D), lambda i: (i, 0)),
            pl.BlockSpec((BLK, D), lambda i: (half + i, 0)),
            pl.BlockSpec((BLK, D), lambda i: (i, 0)),
            pl.BlockSpec((D, D), lambda i: (0, 0)),
            pl.BlockSpec((D, D), lambda i: (0, 0)),
            pl.BlockSpec((1, D), lambda i: (0, 0)),
        ],
        out_specs=pl.BlockSpec((BLK, D), lambda i: (i, 0)),
        out_shape=jax.ShapeDtypeStruct((NPAD, D), F32),
    )


def _decode_call():
    mesh = plsc.VectorSubcoreMesh(core_axis_name="c", subcore_axis_name="s")
    scratch = [
        pltpu.VMEM((1, CHUNK), I32),
        pltpu.VMEM((1, CHUNK), I32),
        pltpu.VMEM((CHUNK, D), F32),
        pltpu.VMEM((CHUNK, D), F32),
        pltpu.VMEM((CHUNK, 16), F32),
        pltpu.SemaphoreType.DMA,
    ]

    def body(z_hbm, src_hbm, dst_hbm, out_hbm, src_v, dst_v, srows, drows,
             acc_v, sem):
        c = lax.axis_index("c")
        s = lax.axis_index("s")
        wid = s * NC + c

        def chunk_body(j, _):
            row = wid * K + j
            off = row * CHUNK
            pltpu.sync_copy(src_hbm.at[pl.ds(row, 1)], src_v)
            pltpu.sync_copy(dst_hbm.at[pl.ds(row, 1)], dst_v)
            pltpu.async_copy(z_hbm.at[src_v.at[0]], srows, sem).wait()
            pltpu.async_copy(z_hbm.at[dst_v.at[0]], drows, sem).wait()

            def edge(e, _):
                a = jnp.zeros((16,), F32)
                for k in range(D // 16):
                    a = a + (srows[e, pl.ds(k * 16, 16)]
                             * drows[e, pl.ds(k * 16, 16)])
                acc_v[e, pl.ds(0, 16)] = a
                return 0
            lax.fori_loop(0, CHUNK, edge, 0)
            pltpu.sync_copy(acc_v, out_hbm.at[pl.ds(off, CHUNK)])
            return 0
        lax.fori_loop(0, K, chunk_body, 0)

    return pl.kernel(body, out_type=jax.ShapeDtypeStruct((E_PAD, 16), F32),
                     mesh=mesh, scratch_types=scratch)


def _rowsum_call():
    BLKE = 4096
    grid = E_PAD // BLKE

    def body(a_ref, out_ref):
        out_ref[...] = jnp.sum(a_ref[...], axis=1)

    return pl.pallas_call(
        body,
        grid=(grid,),
        in_specs=[pl.BlockSpec((BLKE, 16), lambda i: (i, 0))],
        out_specs=pl.BlockSpec((BLKE,), lambda i: (i,)),
        out_shape=jax.ShapeDtypeStruct((E_PAD,), F32),
    )


def kernel(x, edge_index, W1_l, b1_l, W1_r, W2_l, b2_l, W2_r):
    src = edge_index[0]
    dst = edge_index[1]
    pad = E_PAD - E
    src_p = jnp.concatenate([src, jnp.zeros((pad,), I32)]).reshape(NW * K, CHUNK)
    dst_p = jnp.concatenate([dst, jnp.full((pad,), N, I32)]).reshape(NW * K, CHUNK)
    x_p = jnp.zeros((NPAD, D), F32).at[:N].set(x)
    zrows = jnp.zeros((CHUNK, D), F32)
    orows = jnp.ones((CHUNK, D), F32)

    agg1, deg = _segsum_call(True)(x_p, src_p, dst_p, zrows, orows)
    h = _dense_call(True)(agg1, agg1, deg, deg, x_p,
                          W1_l.T, W1_r.T, b1_l[None, :])
    (agg2,) = _segsum_call(False)(h, src_p, dst_p, zrows, orows)
    z = _dense_call(False)(agg2, agg2, deg, deg, h,
                           W2_l.T, W2_r.T, b2_l[None, :])
    partial = _decode_call()(z, src_p, dst_p)
    dots = _rowsum_call()(partial)
    return dots[:E]
